# Initial kernel scaffold; baseline (speedup 1.0000x reference)
import jax
import jax.numpy as jnp
from jax.experimental import pallas as pl


def kernel(points, Wm, b, gamma, beta, mean, var):
    raise NotImplementedError("write your pallas kernel here")


# speedup vs baseline: 1.0414x; 1.0414x over previous
import functools

import jax
import jax.numpy as jnp
from jax.experimental import pallas as pl

PC = (0.0, -40.0, -3.0, 70.4, 40.0, 1.0)
PILLAR = 0.1
H = 800
W = 704
C = 32
NB = 4096


def _mlp_body(pts_ref, wt_ref, a_ref, d_ref, h_ref, seg_ref):
    pts = pts_ref[0]
    x = pts[0:1, :]
    y = pts[1:2, :]
    cxf = jnp.floor((x - PC[0]) / PILLAR)
    cyf = jnp.floor((y - PC[1]) / PILLAR)
    cx = cxf.astype(jnp.int32)
    cy = cyf.astype(jnp.int32)
    mask = (cx >= 0) & (cx < W) & (cy >= 0) & (cy < H)
    cxc = jnp.clip(cx, 0, W - 1)
    cyc = jnp.clip(cy, 0, H - 1)
    center_x = (cxc.astype(jnp.float32) + 0.5) * PILLAR + PC[0]
    center_y = (cyc.astype(jnp.float32) + 0.5) * PILLAR + PC[1]
    dx = x - center_x
    dy = y - center_y
    feats = jnp.concatenate([pts[0:5, :], dx, dy, jnp.zeros_like(dx)], axis=0)
    h = jnp.dot(wt_ref[...], feats, preferred_element_type=jnp.float32)
    h = h * a_ref[...] + d_ref[...]
    h = jnp.maximum(h, 0.0) * mask.astype(jnp.float32)
    h_ref[0] = h
    b = pl.program_id(0)
    seg_ref[0] = b * (H * W) + cyc * W + cxc


def kernel(points, Wm, b, gamma, beta, mean, var):
    B, N, _ = points.shape
    NP = ((N + NB - 1) // NB) * NB
    pts_t = jnp.transpose(points, (0, 2, 1))
    pts_t = jnp.pad(pts_t, ((0, 0), (0, 3), (0, NP - N)), constant_values=-1.0)
    s = gamma / jnp.sqrt(var + 1e-5)
    a_col = s.reshape(C, 1)
    d_col = ((b - mean) * s + beta).reshape(C, 1)
    wt = jnp.pad(Wm, ((0, 1), (0, 0))).T

    grid = (B, NP // NB)
    h_t, seg = pl.pallas_call(
        _mlp_body,
        grid=grid,
        in_specs=[
            pl.BlockSpec((1, 8, NB), lambda bb, i: (bb, 0, i)),
            pl.BlockSpec((C, 8), lambda bb, i: (0, 0)),
            pl.BlockSpec((C, 1), lambda bb, i: (0, 0)),
            pl.BlockSpec((C, 1), lambda bb, i: (0, 0)),
        ],
        out_specs=[
            pl.BlockSpec((1, C, NB), lambda bb, i: (bb, 0, i)),
            pl.BlockSpec((1, 1, NB), lambda bb, i: (bb, 0, i)),
        ],
        out_shape=[
            jax.ShapeDtypeStruct((B, C, NP), jnp.float32),
            jax.ShapeDtypeStruct((B, 1, NP), jnp.int32),
        ],
    )(pts_t, wt, a_col, d_col)

    h_rows = h_t.transpose(0, 2, 1).reshape(B * NP, C)
    seg_flat = seg.reshape(B * NP)
    pooled = jax.ops.segment_max(h_rows, seg_flat, num_segments=B * H * W)
    pooled = jnp.where(jnp.isfinite(pooled), pooled, 0.0)
    return pooled.reshape(B, H, W, C).transpose(0, 3, 1, 2)


# speedup vs baseline: 1.2889x; 1.2376x over previous
import functools

import jax
import jax.numpy as jnp
from jax.experimental import pallas as pl

PC = (0.0, -40.0, -3.0, 70.4, 40.0, 1.0)
PILLAR = 0.1
H = 800
W = 704
C = 32
NB = 4096


def _mlp_body(pts_ref, wt_ref, a_ref, d_ref, h_ref, seg_ref):
    pts = pts_ref[0]
    x = pts[0:1, :]
    y = pts[1:2, :]
    cxf = jnp.floor((x - PC[0]) / PILLAR)
    cyf = jnp.floor((y - PC[1]) / PILLAR)
    cx = cxf.astype(jnp.int32)
    cy = cyf.astype(jnp.int32)
    mask = (cx >= 0) & (cx < W) & (cy >= 0) & (cy < H)
    cxc = jnp.clip(cx, 0, W - 1)
    cyc = jnp.clip(cy, 0, H - 1)
    center_x = (cxc.astype(jnp.float32) + 0.5) * PILLAR + PC[0]
    center_y = (cyc.astype(jnp.float32) + 0.5) * PILLAR + PC[1]
    dx = x - center_x
    dy = y - center_y
    feats = jnp.concatenate([pts[0:5, :], dx, dy, jnp.zeros_like(dx)], axis=0)
    h = jnp.dot(wt_ref[...], feats, preferred_element_type=jnp.float32)
    h = h * a_ref[...] + d_ref[...]
    h = jnp.maximum(h, 0.0) * mask.astype(jnp.float32)
    h_ref[0] = h
    b = pl.program_id(0)
    seg_ref[0] = b * (H * W) + cyc * W + cxc


def kernel(points, Wm, b, gamma, beta, mean, var):
    B, N, _ = points.shape
    NP = ((N + NB - 1) // NB) * NB
    pts_t = jnp.transpose(points, (0, 2, 1))
    pts_t = jnp.pad(pts_t, ((0, 0), (0, 3), (0, NP - N)), constant_values=-1.0)
    s = gamma / jnp.sqrt(var + 1e-5)
    a_col = s.reshape(C, 1)
    d_col = ((b - mean) * s + beta).reshape(C, 1)
    wt = jnp.pad(Wm, ((0, 1), (0, 0))).T

    grid = (B, NP // NB)
    h_t, seg = pl.pallas_call(
        _mlp_body,
        grid=grid,
        in_specs=[
            pl.BlockSpec((1, 8, NB), lambda bb, i: (bb, 0, i)),
            pl.BlockSpec((C, 8), lambda bb, i: (0, 0)),
            pl.BlockSpec((C, 1), lambda bb, i: (0, 0)),
            pl.BlockSpec((C, 1), lambda bb, i: (0, 0)),
        ],
        out_specs=[
            pl.BlockSpec((1, C, NB), lambda bb, i: (bb, 0, i)),
            pl.BlockSpec((1, 1, NB), lambda bb, i: (bb, 0, i)),
        ],
        out_shape=[
            jax.ShapeDtypeStruct((B, C, NP), jnp.float32),
            jax.ShapeDtypeStruct((B, 1, NP), jnp.int32),
        ],
    )(pts_t, wt, a_col, d_col)

    h_rows = h_t.transpose(0, 2, 1).reshape(B * NP, C)
    seg_flat = seg.reshape(B * NP)
    pooled = jnp.zeros((B * H * W, C), jnp.float32).at[seg_flat].max(
        h_rows, mode="promise_in_bounds", unique_indices=False)
    return pooled.reshape(B, H, W, C).transpose(0, 3, 1, 2)
